# baseline (speedup 1.0000x reference)
import functools

import jax
import jax.numpy as jnp
from jax import lax
from jax.experimental import pallas as pl
from jax.experimental.pallas import tpu as pltpu
from jax.experimental.pallas import tpu_sc as plsc

_I, _J = 16384, 200
_V, _D = 10000, 32
_NC, _NS = 2, 16
_NKT = _D // 8
_NE = 8
_CH = _I // _NE
_TI = _CH // 128


@functools.partial(
    pl.kernel,
    mesh=plsc.VectorSubcoreMesh(core_axis_name="c", subcore_axis_name="s"),
    compiler_params=pltpu.CompilerParams(
        use_tc_tiling_on_sc=False, needs_layout_passes=False
    ),
    out_type=jax.ShapeDtypeStruct((_J, _NKT, _I // 128, 8, 128), jnp.float32),
    scratch_types=(
        [pltpu.VMEM((_V,), jnp.float32) for _ in range(8)]
        + [
            pltpu.VMEM((2, _CH), jnp.int32),
            pltpu.VMEM((2, _TI, 8, 128), jnp.float32),
            pltpu.SemaphoreType.DMA,
            pltpu.SemaphoreType.DMA,
            pltpu.SemaphoreType.DMA,
            pltpu.SemaphoreType.DMA,
        ]
    ),
)
def _sc_lookup(tableT_hbm, idxT_hbm, out_hbm,
               t0, t1, t2, t3, t4, t5, t6, t7,
               idx_v, out_v, sem_i0, sem_i1, sem_o0, sem_o1):
    tbls = (t0, t1, t2, t3, t4, t5, t6, t7)
    sems_i = (sem_i0, sem_i1)
    sems_o = (sem_o0, sem_o1)
    w = lax.axis_index("s") * _NC + lax.axis_index("c")
    ktile = w % _NKT
    e = w // _NKT
    ibase = e * _CH

    tcopies = [
        pltpu.async_copy(tableT_hbm.at[ktile * 8 + d], tbls[d], sem_o0)
        for d in range(8)
    ]
    pltpu.async_copy(idxT_hbm.at[0, pl.ds(ibase, _CH)], idx_v.at[0], sems_i[0])
    pltpu.async_copy(idxT_hbm.at[1, pl.ds(ibase, _CH)], idx_v.at[1], sems_i[1])
    for c in tcopies:
        c.wait()

    def relu_body(i, carry):
        for d in range(8):
            tbls[d][pl.ds(i * 16, 16)] = jnp.maximum(tbls[d][pl.ds(i * 16, 16)], 0.0)
        return carry

    lax.fori_loop(0, _V // 16, relu_body, 0)

    def body(j2, carry):
        for b in range(2):
            j = j2 * 2 + b
            pltpu.make_async_copy(
                idxT_hbm.at[0, pl.ds(0, _CH)], idx_v.at[b], sems_i[b]
            ).wait()

            @pl.when(j2 > 0)
            def _drain():
                pltpu.make_async_copy(
                    out_v.at[b], out_hbm.at[0, 0, pl.ds(0, _TI)], sems_o[b]
                ).wait()

            @plsc.parallel_loop(0, _TI, 1, unroll=4)
            def ibody(r):
                base = r * 128
                for u in range(8):
                    vec = idx_v[b, pl.ds(base + u * 16, 16)]
                    for d in range(8):
                        out_v[b, r, d, pl.ds(u * 16, 16)] = plsc.load_gather(
                            tbls[d], [vec]
                        )

            pltpu.async_copy(
                out_v.at[b],
                out_hbm.at[j, ktile, pl.ds(e * _TI, _TI)],
                sems_o[b],
            )

            jn = j + 2

            @pl.when(jn < _J)
            def _prefetch():
                pltpu.async_copy(
                    idxT_hbm.at[jn, pl.ds(ibase, _CH)], idx_v.at[b], sems_i[b]
                )

        return carry

    lax.fori_loop(0, _J // 2, body, 0)

    for b in range(2):
        pltpu.make_async_copy(
            out_v.at[b], out_hbm.at[0, 0, pl.ds(0, _TI)], sems_o[b]
        ).wait()


def kernel(x, kernel):
    idxT = jnp.transpose(x.astype(jnp.int32))
    tableT = jnp.transpose(kernel)
    out5 = _sc_lookup(tableT, idxT)
    return jnp.transpose(out5, (2, 4, 0, 1, 3)).reshape(_I, _J, _D)


# speedup vs baseline: 1.0910x; 1.0910x over previous
import functools

import jax
import jax.numpy as jnp
from jax import lax
from jax.experimental import pallas as pl
from jax.experimental.pallas import tpu as pltpu
from jax.experimental.pallas import tpu_sc as plsc

_I, _J = 16384, 200
_V, _D = 10000, 32
_NC, _NS = 2, 16
_NKT = _D // 8
_NE = 8
_CH = _I // _NE
_TI = _CH // 128


@functools.partial(
    pl.kernel,
    mesh=plsc.VectorSubcoreMesh(core_axis_name="c", subcore_axis_name="s"),
    compiler_params=pltpu.CompilerParams(
        use_tc_tiling_on_sc=False, needs_layout_passes=False
    ),
    out_type=jax.ShapeDtypeStruct((_J, _NKT, _I // 128, 8, 128), jnp.float32),
    scratch_types=(
        [pltpu.VMEM((_V,), jnp.float32) for _ in range(8)]
        + [
            pltpu.VMEM((2, _CH), jnp.int32),
            pltpu.VMEM((2, _TI, 8, 128), jnp.float32),
            pltpu.SemaphoreType.DMA,
            pltpu.SemaphoreType.DMA,
            pltpu.SemaphoreType.DMA,
            pltpu.SemaphoreType.DMA,
        ]
    ),
)
def _sc_lookup(tableT_hbm, idxT_hbm, out_hbm,
               t0, t1, t2, t3, t4, t5, t6, t7,
               idx_v, out_v, sem_i0, sem_i1, sem_o0, sem_o1):
    tbls = (t0, t1, t2, t3, t4, t5, t6, t7)
    sems_i = (sem_i0, sem_i1)
    sems_o = (sem_o0, sem_o1)
    w = lax.axis_index("s") * _NC + lax.axis_index("c")
    ktile = w % _NKT
    e = w // _NKT
    ibase = e * _CH

    tcopies = [
        pltpu.async_copy(tableT_hbm.at[ktile * 8 + d], tbls[d], sem_o0)
        for d in range(8)
    ]
    pltpu.async_copy(idxT_hbm.at[0, pl.ds(ibase, _CH)], idx_v.at[0], sems_i[0])
    pltpu.async_copy(idxT_hbm.at[1, pl.ds(ibase, _CH)], idx_v.at[1], sems_i[1])
    for c in tcopies:
        c.wait()

    def relu_body(i, carry):
        for d in range(8):
            tbls[d][pl.ds(i * 16, 16)] = jnp.maximum(tbls[d][pl.ds(i * 16, 16)], 0.0)
        return carry

    lax.fori_loop(0, _V // 16, relu_body, 0)

    def body(j2, carry):
        for b in range(2):
            j = j2 * 2 + b
            pltpu.make_async_copy(
                idxT_hbm.at[0, pl.ds(0, _CH)], idx_v.at[b], sems_i[b]
            ).wait()

            @pl.when(j2 > 0)
            def _drain():
                pltpu.make_async_copy(
                    out_v.at[b], out_hbm.at[0, 0, pl.ds(0, _TI)], sems_o[b]
                ).wait()

            @plsc.parallel_loop(0, _TI, 1, unroll=2)
            def ibody(r):
                base = r * 128
                for u in range(8):
                    vec = idx_v[b, pl.ds(base + u * 16, 16)]
                    for d in range(8):
                        out_v[b, r, d, pl.ds(u * 16, 16)] = plsc.load_gather(
                            tbls[d], [vec]
                        )

            pltpu.async_copy(
                out_v.at[b],
                out_hbm.at[j, ktile, pl.ds(e * _TI, _TI)],
                sems_o[b],
            )

            jn = j + 2

            @pl.when(jn < _J)
            def _prefetch():
                pltpu.async_copy(
                    idxT_hbm.at[jn, pl.ds(ibase, _CH)], idx_v.at[b], sems_i[b]
                )

        return carry

    lax.fori_loop(0, _J // 2, body, 0)

    for b in range(2):
        pltpu.make_async_copy(
            out_v.at[b], out_hbm.at[0, 0, pl.ds(0, _TI)], sems_o[b]
        ).wait()


def kernel(x, kernel):
    idxT = jnp.transpose(x.astype(jnp.int32))
    tableT = jnp.transpose(kernel)
    out5 = _sc_lookup(tableT, idxT)
    return jnp.transpose(out5, (2, 4, 0, 1, 3)).reshape(_I, _J, _D)
